# baseline (speedup 1.0000x reference)
import jax
import jax.numpy as jnp
from jax import lax
from jax.experimental import pallas as pl
from jax.experimental.pallas import tpu as pltpu
from jax.experimental.pallas import tpu_sc as plsc

_NUM_CLASSES = 1000
_DIM = 128
_N = 320000
_MOMENTUM = 0.99

_NC = 2
_NS = 16
_NW = _NC * _NS
_G = 128
_NG = _N // _G
_CPAD = 1024
_CPS = _CPAD // _NS


def _phase1_body(emb_hbm, lab_hbm, labp_hbm, zero_hbm, zero1k_hbm,
                 sums_out, a_out, b_out,
                 rows_v, lab_v, labp_v, stage_v, a_v, b_v, sums_sp,
                 sem0, sem1, ssem0, ssem1):
    c = lax.axis_index("c")
    s = lax.axis_index("s")
    wid = s * _NC + c

    base = s * _CPS
    pltpu.sync_copy(zero_hbm, stage_v)
    pltpu.sync_copy(stage_v, sums_sp.at[pl.ds(base, _CPS)])
    pltpu.sync_copy(zero1k_hbm, a_v)
    pltpu.sync_copy(zero1k_hbm, b_v)
    plsc.subcore_barrier()

    g0 = wid * _NG // _NW
    g1 = (wid + 1) * _NG // _NW
    sems = (sem0, sem1)
    ssems = (ssem0, ssem1)

    def _start_load(g, b, sem):
        pltpu.async_copy(emb_hbm.at[pl.ds(g * _G, _G)], rows_v.at[b], sem)
        pltpu.async_copy(lab_hbm.at[g], lab_v.at[b], sem)
        pltpu.async_copy(labp_hbm.at[g], labp_v.at[b], sem)

    def _wait_load(g, b, sem):
        pltpu.make_async_copy(emb_hbm.at[pl.ds(g * _G, _G)],
                              rows_v.at[b], sem).wait()
        pltpu.make_async_copy(lab_hbm.at[g], lab_v.at[b], sem).wait()
        pltpu.make_async_copy(labp_hbm.at[g], labp_v.at[b], sem).wait()

    pl.when(g0 < g1)(lambda: _start_load(g0, 0, sem0))
    pl.when(g0 + 1 < g1)(lambda: _start_load(g0 + 1, 1, sem1))

    iota16 = lax.iota(jnp.int32, 16)

    def _fire(g, b):
        _wait_load(g, b, sems[b])
        pltpu.async_copy(
            rows_v.at[b], sums_sp.at[lab_v.at[b]], ssems[b], add=True)

    def _count(g, b):
        for k in range(8):
            l = lab_v[b, pl.ds(16 * k, 16)]
            lp = labp_v[b, pl.ds(16 * k, 16)]
            bmask = l != lp
            amask = jnp.logical_and(bmask, lp >= 0)
            posf = (iota16 + (g * _G + 16 * k + 1)).astype(jnp.float32)
            plsc.addupdate_scatter(b_v, [l], posf, mask=bmask)
            plsc.addupdate_scatter(a_v, [lp], posf, mask=amask)

    def _drain(g, b):
        pltpu.make_async_copy(
            rows_v.at[b], sums_sp.at[lab_v.at[b]], ssems[b]).wait()
        pl.when(g + 2 < g1)(lambda: _start_load(g + 2, b, sems[b]))

    def _pair(p, _):
        ga = g0 + 2 * p
        gb = ga + 1
        pl.when(ga < g1)(lambda: _fire(ga, 0))
        pl.when(gb < g1)(lambda: _fire(gb, 1))
        pl.when(ga < g1)(lambda: _count(ga, 0))
        pl.when(gb < g1)(lambda: _count(gb, 1))
        pl.when(ga < g1)(lambda: _drain(ga, 0))
        pl.when(gb < g1)(lambda: _drain(gb, 1))
        return 0
    lax.fori_loop(0, (g1 - g0 + 1) // 2, _pair, 0)

    plsc.subcore_barrier()

    pltpu.sync_copy(sums_sp.at[pl.ds(base, _CPS)], stage_v)
    pltpu.sync_copy(stage_v, sums_out.at[c, pl.ds(base, _CPS)])
    pltpu.sync_copy(a_v, a_out.at[wid])
    pltpu.sync_copy(b_v, b_out.at[wid])


def _phase1(embeddings, lab2d, labp2d):
    zero = jnp.zeros((_CPS, _DIM), jnp.float32)
    zero1k = jnp.zeros((_CPAD,), jnp.float32)
    return _phase1_call(embeddings, lab2d, labp2d, zero, zero1k)


_phase1_call = pl.kernel(
    _phase1_body,
    out_type=(
        jax.ShapeDtypeStruct((_NC, _CPAD, _DIM), jnp.float32),
        jax.ShapeDtypeStruct((_NW, _CPAD), jnp.float32),
        jax.ShapeDtypeStruct((_NW, _CPAD), jnp.float32),
    ),
    mesh=plsc.VectorSubcoreMesh(
        core_axis_name="c", subcore_axis_name="s",
        num_cores=_NC, num_subcores=_NS),
    compiler_params=pltpu.CompilerParams(needs_layout_passes=False),
    scratch_types=(
        pltpu.VMEM((2, _G, _DIM), jnp.float32),
        pltpu.VMEM((2, _G), jnp.int32),
        pltpu.VMEM((2, _G), jnp.int32),
        pltpu.VMEM((_CPS, _DIM), jnp.float32),
        pltpu.VMEM((_CPAD,), jnp.float32),
        pltpu.VMEM((_CPAD,), jnp.float32),
        pltpu.VMEM_SHARED((_CPAD, _DIM), jnp.float32),
        pltpu.SemaphoreType.DMA,
        pltpu.SemaphoreType.DMA,
        pltpu.SemaphoreType.DMA,
        pltpu.SemaphoreType.DMA,
    ),
)


def _combine_body(sums_ref, a_ref, b_ref, proto_ref, out_ref):
    total = sums_ref[0] + sums_ref[1]
    a = jnp.sum(a_ref[...], axis=0)
    bb = jnp.sum(b_ref[...], axis=0)
    cnt = jnp.where(a > 0.0, a - bb, (_N + 1.0) - bb)
    cnt = jnp.where(bb > 0.0, cnt, 0.0)
    mean = total / jnp.maximum(cnt, 1.0)
    proto = proto_ref[...]
    out_ref[...] = jnp.where(
        cnt > 0.0, _MOMENTUM * proto + (1.0 - _MOMENTUM) * mean, proto)


def kernel(embeddings, labels, prototypes):
    lab = labels.astype(jnp.int32)
    lab2d = lab.reshape(_NG, _G)
    labp2d = jnp.concatenate(
        [jnp.full((1,), -1, jnp.int32), lab[:-1]]).reshape(_NG, _G)
    sums, a, b = _phase1(embeddings, lab2d, labp2d)
    a3 = a[:, :_NUM_CLASSES].reshape(_NW, _NUM_CLASSES, 1)
    b3 = b[:, :_NUM_CLASSES].reshape(_NW, _NUM_CLASSES, 1)
    return pl.pallas_call(
        _combine_body,
        out_shape=jax.ShapeDtypeStruct((_NUM_CLASSES, _DIM), jnp.float32),
    )(sums[:, :_NUM_CLASSES], a3, b3, prototypes)


# speedup vs baseline: 1.2441x; 1.2441x over previous
import jax
import jax.numpy as jnp
from jax import lax
from jax.experimental import pallas as pl
from jax.experimental.pallas import tpu as pltpu
from jax.experimental.pallas import tpu_sc as plsc

_NUM_CLASSES = 1000
_DIM = 128
_N = 320000
_MOMENTUM = 0.99

_NC = 2
_NS = 16
_NW = _NC * _NS
_G = 128
_NG = _N // _G
_CPAD = 1024
_CPS = _CPAD // _NS


def _phase1_body(emb_hbm, lab_hbm, labp_hbm, zero_hbm, zero1k_hbm,
                 sums_out, a_out, b_out,
                 rows_v, lab_v, labp_v, stage_v, a_v, b_v, sums_sp,
                 sem0, sem1, ssem0, ssem1):
    c = lax.axis_index("c")
    s = lax.axis_index("s")
    wid = s * _NC + c

    base = s * _CPS
    pltpu.sync_copy(zero_hbm, stage_v)
    pltpu.sync_copy(stage_v, sums_sp.at[pl.ds(base, _CPS)])
    pltpu.sync_copy(zero1k_hbm, a_v)
    pltpu.sync_copy(zero1k_hbm, b_v)
    plsc.subcore_barrier()

    g0 = wid * _NG // _NW
    g1 = (wid + 1) * _NG // _NW
    sems = (sem0, sem1)
    ssems = (ssem0, ssem1)

    def _start_load(g, b, sem):
        pltpu.async_copy(emb_hbm.at[pl.ds(g * _G, _G)], rows_v.at[b], sem)
        pltpu.async_copy(lab_hbm.at[g], lab_v.at[b], sem)
        pltpu.async_copy(labp_hbm.at[g], labp_v.at[b], sem)

    def _wait_load(g, b, sem):
        pltpu.make_async_copy(emb_hbm.at[pl.ds(g * _G, _G)],
                              rows_v.at[b], sem).wait()
        pltpu.make_async_copy(lab_hbm.at[g], lab_v.at[b], sem).wait()
        pltpu.make_async_copy(labp_hbm.at[g], labp_v.at[b], sem).wait()

    pl.when(g0 < g1)(lambda: _start_load(g0, 0, sem0))
    pl.when(g0 + 1 < g1)(lambda: _start_load(g0 + 1, 1, sem1))

    iota16 = lax.iota(jnp.int32, 16)

    def _fire(g, b):
        _wait_load(g, b, sems[b])
        pltpu.async_copy(
            rows_v.at[b], sums_sp.at[lab_v.at[b]], ssems[b], add=True)

    def _count(g, b):
        for k in range(8):
            l = lab_v[b, pl.ds(16 * k, 16)]
            lp = labp_v[b, pl.ds(16 * k, 16)]
            bmask = l != lp
            amask = jnp.logical_and(bmask, lp >= 0)
            posf = (iota16 + (g * _G + 16 * k + 1)).astype(jnp.float32)
            plsc.addupdate_scatter(b_v, [l], posf, mask=bmask)
            plsc.addupdate_scatter(a_v, [lp], posf, mask=amask)

    def _drain(g, b):
        pltpu.make_async_copy(
            rows_v.at[b], sums_sp.at[lab_v.at[b]], ssems[b]).wait()
        pl.when(g + 2 < g1)(lambda: _start_load(g + 2, b, sems[b]))

    def _pair(p, _):
        for b in range(2):
            g = g0 + 2 * p + b

            def _do(g=g, b=b):
                _fire(g, b)
                _count(g, b)
                _drain(g, b)

            pl.when(g < g1)(_do)
        return 0
    lax.fori_loop(0, (g1 - g0 + 1) // 2, _pair, 0)

    plsc.subcore_barrier()

    pltpu.sync_copy(sums_sp.at[pl.ds(base, _CPS)], stage_v)
    pltpu.sync_copy(stage_v, sums_out.at[c, pl.ds(base, _CPS)])
    pltpu.sync_copy(a_v, a_out.at[wid])
    pltpu.sync_copy(b_v, b_out.at[wid])


def _phase1(embeddings, lab2d, labp2d):
    zero = jnp.zeros((_CPS, _DIM), jnp.float32)
    zero1k = jnp.zeros((_CPAD,), jnp.float32)
    return _phase1_call(embeddings, lab2d, labp2d, zero, zero1k)


_phase1_call = pl.kernel(
    _phase1_body,
    out_type=(
        jax.ShapeDtypeStruct((_NC, _CPAD, _DIM), jnp.float32),
        jax.ShapeDtypeStruct((_NW, _CPAD), jnp.float32),
        jax.ShapeDtypeStruct((_NW, _CPAD), jnp.float32),
    ),
    mesh=plsc.VectorSubcoreMesh(
        core_axis_name="c", subcore_axis_name="s",
        num_cores=_NC, num_subcores=_NS),
    compiler_params=pltpu.CompilerParams(needs_layout_passes=False),
    scratch_types=(
        pltpu.VMEM((2, _G, _DIM), jnp.float32),
        pltpu.VMEM((2, _G), jnp.int32),
        pltpu.VMEM((2, _G), jnp.int32),
        pltpu.VMEM((_CPS, _DIM), jnp.float32),
        pltpu.VMEM((_CPAD,), jnp.float32),
        pltpu.VMEM((_CPAD,), jnp.float32),
        pltpu.VMEM_SHARED((_CPAD, _DIM), jnp.float32),
        pltpu.SemaphoreType.DMA,
        pltpu.SemaphoreType.DMA,
        pltpu.SemaphoreType.DMA,
        pltpu.SemaphoreType.DMA,
    ),
)


def _combine_body(sums_ref, a_ref, b_ref, proto_ref, out_ref):
    total = sums_ref[0] + sums_ref[1]
    a = jnp.sum(a_ref[...], axis=0)
    bb = jnp.sum(b_ref[...], axis=0)
    cnt = jnp.where(a > 0.0, a - bb, (_N + 1.0) - bb)
    cnt = jnp.where(bb > 0.0, cnt, 0.0)
    mean = total / jnp.maximum(cnt, 1.0)
    proto = proto_ref[...]
    out_ref[...] = jnp.where(
        cnt > 0.0, _MOMENTUM * proto + (1.0 - _MOMENTUM) * mean, proto)


def kernel(embeddings, labels, prototypes):
    lab = labels.astype(jnp.int32)
    lab2d = lab.reshape(_NG, _G)
    labp2d = jnp.concatenate(
        [jnp.full((1,), -1, jnp.int32), lab[:-1]]).reshape(_NG, _G)
    sums, a, b = _phase1(embeddings, lab2d, labp2d)
    a3 = a[:, :_NUM_CLASSES].reshape(_NW, _NUM_CLASSES, 1)
    b3 = b[:, :_NUM_CLASSES].reshape(_NW, _NUM_CLASSES, 1)
    return pl.pallas_call(
        _combine_body,
        out_shape=jax.ShapeDtypeStruct((_NUM_CLASSES, _DIM), jnp.float32),
    )(sums[:, :_NUM_CLASSES], a3, b3, prototypes)


# speedup vs baseline: 1.4598x; 1.1734x over previous
import jax
import jax.numpy as jnp
from jax import lax
from jax.experimental import pallas as pl
from jax.experimental.pallas import tpu as pltpu
from jax.experimental.pallas import tpu_sc as plsc

_NUM_CLASSES = 1000
_DIM = 128
_N = 320000
_MOMENTUM = 0.99

_NC = 2
_NS = 16
_NW = _NC * _NS
_G = 128
_NG = _N // _G
_CPAD = 1024
_CPS = _CPAD // _NS


def _phase1_body(emb_hbm, lab_hbm, labp_hbm, zero_hbm, zero1k_hbm,
                 sums_out, a_out, b_out,
                 rows_v, lab_v, labp_v, stage_v, a_v, b_v, sums_sp,
                 sem0, sem1, ssem0, ssem1):
    c = lax.axis_index("c")
    s = lax.axis_index("s")
    wid = s * _NC + c

    base = s * _CPS
    pltpu.sync_copy(zero_hbm, stage_v)
    pltpu.sync_copy(stage_v, sums_sp.at[pl.ds(base, _CPS)])
    pltpu.sync_copy(zero1k_hbm, a_v)
    pltpu.sync_copy(zero1k_hbm, b_v)
    plsc.subcore_barrier()

    g0 = wid * _NG // _NW
    g1 = (wid + 1) * _NG // _NW
    sems = (sem0, sem1)
    ssems = (ssem0, ssem1)

    def _start_load(g, b, sem):
        pltpu.async_copy(emb_hbm.at[pl.ds(g * _G, _G)], rows_v.at[b], sem)
        pltpu.async_copy(lab_hbm.at[g], lab_v.at[b], sem)
        pltpu.async_copy(labp_hbm.at[g], labp_v.at[b], sem)

    def _wait_load(g, b, sem):
        pltpu.make_async_copy(emb_hbm.at[pl.ds(g * _G, _G)],
                              rows_v.at[b], sem).wait()
        pltpu.make_async_copy(lab_hbm.at[g], lab_v.at[b], sem).wait()
        pltpu.make_async_copy(labp_hbm.at[g], labp_v.at[b], sem).wait()

    pl.when(g0 < g1)(lambda: _start_load(g0, 0, sem0))
    pl.when(g0 + 1 < g1)(lambda: _start_load(g0 + 1, 1, sem1))

    iota16 = lax.iota(jnp.int32, 16)

    def _fire(g, b):
        _wait_load(g, b, sems[b])
        pltpu.async_copy(
            rows_v.at[b], sums_sp.at[lab_v.at[b]], ssems[b], add=True)

    def _count(g, b):
        for k in range(8):
            l = lab_v[b, pl.ds(16 * k, 16)]
            lp = labp_v[b, pl.ds(16 * k, 16)]
            bmask = l != lp
            amask = jnp.logical_and(bmask, lp >= 0)
            posf = (iota16 + (g * _G + 16 * k + 1)).astype(jnp.float32)
            pidx = ((l & 7) << 7) + (l >> 3)
            ppidx = ((lp & 7) << 7) + (lp >> 3)
            plsc.addupdate_scatter(b_v, [pidx], posf, mask=bmask)
            plsc.addupdate_scatter(a_v, [ppidx], posf, mask=amask)

    def _drain(g, b):
        pltpu.make_async_copy(
            rows_v.at[b], sums_sp.at[lab_v.at[b]], ssems[b]).wait()
        pl.when(g + 2 < g1)(lambda: _start_load(g + 2, b, sems[b]))

    def _pair(p, _):
        for b in range(2):
            g = g0 + 2 * p + b

            def _do(g=g, b=b):
                _fire(g, b)
                _count(g, b)
                _drain(g, b)

            pl.when(g < g1)(_do)
        return 0
    lax.fori_loop(0, (g1 - g0 + 1) // 2, _pair, 0)

    plsc.subcore_barrier()

    pltpu.sync_copy(sums_sp.at[pl.ds(base, _CPS)], stage_v)
    pltpu.sync_copy(stage_v, sums_out.at[c, pl.ds(base, _CPS)])
    pltpu.sync_copy(a_v, a_out.at[wid])
    pltpu.sync_copy(b_v, b_out.at[wid])


def _phase1(embeddings, lab2d, labp2d):
    zero = jnp.zeros((_CPS, _DIM), jnp.float32)
    zero1k = jnp.zeros((_CPAD,), jnp.float32)
    return _phase1_call(embeddings, lab2d, labp2d, zero, zero1k)


_phase1_call = pl.kernel(
    _phase1_body,
    out_type=(
        jax.ShapeDtypeStruct((_NC, _CPAD, _DIM), jnp.float32),
        jax.ShapeDtypeStruct((_NW, _CPAD), jnp.float32),
        jax.ShapeDtypeStruct((_NW, _CPAD), jnp.float32),
    ),
    mesh=plsc.VectorSubcoreMesh(
        core_axis_name="c", subcore_axis_name="s",
        num_cores=_NC, num_subcores=_NS),
    compiler_params=pltpu.CompilerParams(needs_layout_passes=False),
    scratch_types=(
        pltpu.VMEM((2, _G, _DIM), jnp.float32),
        pltpu.VMEM((2, _G), jnp.int32),
        pltpu.VMEM((2, _G), jnp.int32),
        pltpu.VMEM((_CPS, _DIM), jnp.float32),
        pltpu.VMEM((_CPAD,), jnp.float32),
        pltpu.VMEM((_CPAD,), jnp.float32),
        pltpu.VMEM_SHARED((_CPAD, _DIM), jnp.float32),
        pltpu.SemaphoreType.DMA,
        pltpu.SemaphoreType.DMA,
        pltpu.SemaphoreType.DMA,
        pltpu.SemaphoreType.DMA,
    ),
)


def _combine_body(sums_ref, a_ref, b_ref, rsel_ref, csel_ref, ones_ref,
                  proto_ref, out_ref):
    total = sums_ref[0] + sums_ref[1]
    asum = jnp.sum(a_ref[...], axis=0)
    bsum = jnp.sum(b_ref[...], axis=0)
    rsel = rsel_ref[...]
    csel = csel_ref[...]
    ones = ones_ref[...]
    dot = lambda x, y: jax.lax.dot(
        x, y, preferred_element_type=jnp.float32,
        precision=jax.lax.Precision.HIGHEST)
    abc = dot(dot(rsel, asum) * csel, ones)
    bbc = dot(dot(rsel, bsum) * csel, ones)
    cnt = jnp.where(abc > 0.0, abc - bbc, (_N + 1.0) - bbc)
    cnt = jnp.where(bbc > 0.0, cnt, 0.0)
    mean = total / jnp.maximum(cnt, 1.0)
    proto = proto_ref[...]
    out_ref[...] = jnp.where(
        cnt > 0.0, _MOMENTUM * proto + (1.0 - _MOMENTUM) * mean, proto)


def kernel(embeddings, labels, prototypes):
    lab = labels.astype(jnp.int32)
    lab2d = lab.reshape(_NG, _G)
    labp2d = jnp.concatenate(
        [jnp.full((1,), -1, jnp.int32), lab[:-1]]).reshape(_NG, _G)
    sums, a, b = _phase1(embeddings, lab2d, labp2d)
    a3 = a.reshape(_NW, 8, _DIM)
    b3 = b.reshape(_NW, 8, _DIM)
    cls = jnp.arange(_NUM_CLASSES)
    rsel = (cls[:, None] % 8 == jnp.arange(8)[None, :]).astype(jnp.float32)
    csel = (cls[:, None] // 8 == jnp.arange(_DIM)[None, :]).astype(jnp.float32)
    ones = jnp.ones((_DIM, _DIM), jnp.float32)
    return pl.pallas_call(
        _combine_body,
        out_shape=jax.ShapeDtypeStruct((_NUM_CLASSES, _DIM), jnp.float32),
    )(sums[:, :_NUM_CLASSES], a3, b3, rsel, csel, ones, prototypes)
